# baseline (speedup 1.0000x reference)
import functools

import jax
import jax.numpy as jnp
from jax import lax
from jax.experimental import pallas as pl
from jax.experimental.pallas import tpu as pltpu
from jax.experimental.pallas import tpu_sc as plsc


def _build_gather(n_total, d, chunk, nbuf):
    info = plsc.get_sparse_core_info()
    num_workers = info.num_cores * info.num_subcores
    n_per_w = n_total // num_workers
    n_chunks = n_per_w // chunk
    n_groups = n_chunks // nbuf
    assert n_total % num_workers == 0
    assert n_per_w % chunk == 0 and n_chunks % nbuf == 0

    mesh = plsc.VectorSubcoreMesh(core_axis_name="c", subcore_axis_name="s")

    @functools.partial(
        pl.kernel,
        mesh=mesh,
        out_type=jax.ShapeDtypeStruct((n_total, d), jnp.float32),
        scratch_types=[
            pltpu.VMEM((n_per_w,), jnp.int32),
            pltpu.VMEM((nbuf, chunk, d), jnp.float32),
        ]
        + [pltpu.SemaphoreType.DMA] * (2 * nbuf),
    )
    def gather_kernel(ids_hbm, table_hbm, out_hbm, idx_v, rows_v, *sems):
        gsem = sems[:nbuf]
        wsem = sems[nbuf:]
        wid = lax.axis_index("s") * info.num_cores + lax.axis_index("c")
        base = wid * n_per_w
        pltpu.sync_copy(ids_hbm.at[pl.ds(base, n_per_w)], idx_v)

        def fire_gather(c, b):
            off = pl.multiple_of(c * chunk, chunk)
            pltpu.async_copy(
                table_hbm.at[idx_v.at[pl.ds(off, chunk)]], rows_v.at[b], gsem[b]
            )

        def fire_write(c, b):
            off = pl.multiple_of(c * chunk, chunk)
            pltpu.async_copy(
                rows_v.at[b], out_hbm.at[pl.ds(base + off, chunk)], wsem[b]
            )

        for b in range(nbuf):
            fire_gather(b, b)

        def body(g, carry):
            for b in range(nbuf):
                c = g * nbuf + b
                pltpu.make_async_copy(
                    table_hbm.at[idx_v.at[pl.ds(0, chunk)]], rows_v.at[b], gsem[b]
                ).wait()
                fire_write(c, b)
            for b in range(nbuf):
                pltpu.make_async_copy(
                    rows_v.at[b],
                    out_hbm.at[pl.ds(base, chunk)],
                    wsem[b],
                ).wait()

                @pl.when(g + 1 < n_groups)
                def _():
                    fire_gather((g + 1) * nbuf + b, b)

            return carry

        lax.fori_loop(0, n_groups, body, 0)

    return gather_kernel


def kernel(input_ids, table):
    b, s = input_ids.shape
    v, d = table.shape
    n_total = b * s
    flat_ids = input_ids.T.reshape(n_total).astype(jnp.int32)
    gather = _build_gather(n_total, d, chunk=160, nbuf=5)
    out = gather(flat_ids, table)
    return out.reshape(s, b, d).transpose(1, 0, 2)


# speedup vs baseline: 1.0303x; 1.0303x over previous
import functools

import jax
import jax.numpy as jnp
from jax import lax
from jax.experimental import pallas as pl
from jax.experimental.pallas import tpu as pltpu
from jax.experimental.pallas import tpu_sc as plsc


def _build_gather(n_total, d, chunk, nbuf):
    info = plsc.get_sparse_core_info()
    num_workers = info.num_cores * info.num_subcores
    n_per_w = n_total // num_workers
    n_chunks = n_per_w // chunk
    n_groups = n_chunks // nbuf
    assert n_total % num_workers == 0
    assert n_per_w % chunk == 0 and n_chunks % nbuf == 0

    mesh = plsc.VectorSubcoreMesh(core_axis_name="c", subcore_axis_name="s")

    @functools.partial(
        pl.kernel,
        mesh=mesh,
        out_type=jax.ShapeDtypeStruct((n_total, d), jnp.float32),
        scratch_types=[
            pltpu.VMEM((n_per_w,), jnp.int32),
            pltpu.VMEM((nbuf, chunk, d), jnp.float32),
        ]
        + [pltpu.SemaphoreType.DMA] * (2 * nbuf),
    )
    def gather_kernel(ids_hbm, table_hbm, out_hbm, idx_v, rows_v, *sems):
        gsem = sems[:nbuf]
        wsem = sems[nbuf:]
        wid = lax.axis_index("s") * info.num_cores + lax.axis_index("c")
        base = wid * n_per_w
        pltpu.sync_copy(ids_hbm.at[pl.ds(base, n_per_w)], idx_v)

        def fire_gather(c, b):
            off = pl.multiple_of(c * chunk, chunk)
            pltpu.async_copy(
                table_hbm.at[idx_v.at[pl.ds(off, chunk)]], rows_v.at[b], gsem[b]
            )

        def fire_write(c, b):
            off = pl.multiple_of(c * chunk, chunk)
            pltpu.async_copy(
                rows_v.at[b], out_hbm.at[pl.ds(base + off, chunk)], wsem[b]
            )

        for b in range(nbuf):
            fire_gather(b, b)

        def body(g, carry):
            for b in range(nbuf):
                c = g * nbuf + b
                pltpu.make_async_copy(
                    table_hbm.at[idx_v.at[pl.ds(0, chunk)]], rows_v.at[b], gsem[b]
                ).wait()
                fire_write(c, b)
            for b in range(nbuf):
                pltpu.make_async_copy(
                    rows_v.at[b],
                    out_hbm.at[pl.ds(base, chunk)],
                    wsem[b],
                ).wait()

                @pl.when(g + 1 < n_groups)
                def _():
                    fire_gather((g + 1) * nbuf + b, b)

            return carry

        lax.fori_loop(0, n_groups, body, 0)

    return gather_kernel


def kernel(input_ids, table):
    b, s = input_ids.shape
    v, d = table.shape
    n_total = b * s
    flat_ids = input_ids.T.reshape(n_total).astype(jnp.int32)
    gather = _build_gather(n_total, d, chunk=64, nbuf=10)
    out = gather(flat_ids, table)
    return out.reshape(s, b, d).transpose(1, 0, 2)
